# baseline (speedup 1.0000x reference)
import jax
import jax.numpy as jnp
from jax import lax
from jax.experimental import pallas as pl
from jax.experimental.pallas import tpu as pltpu
from jax.experimental.pallas import tpu_sc as plsc

VOCAB = 1000000
EMB_DIM = 64
MAX_LEN = 200
BATCH = 4096
SEQ = 200

NUM_CORES = 2
NUM_SUBCORES = 16
NUM_WORKERS = NUM_CORES * NUM_SUBCORES
BBLK = BATCH // NUM_WORKERS
LANES = 16
VECS_PER_ROW = EMB_DIM // LANES
NBUF = 4


def _trig_table(dim, max_len):
    len_size = jnp.tile(jnp.arange(max_len)[:, None], (1, dim)).astype(jnp.float32)
    dim_scale = jnp.power(10000.0, jnp.arange(dim).astype(jnp.float32) / dim)
    pos_s = jnp.sin(len_size / dim_scale)
    pos_c = jnp.cos(len_size / dim_scale)
    out = jnp.concatenate((pos_s, pos_c), axis=0)
    return out.reshape(max_len, -1)


def _body(x_hbm, trig_hbm, table_hbm, out_hbm,
          idx_v, trig_v, r0, r1, r2, r3, gs0, gs1, gs2, gs3,
          ss0, ss1, ss2, ss3):
    c = lax.axis_index("c")
    s_ax = lax.axis_index("s")
    wid = s_ax * NUM_CORES + c
    b0 = wid * BBLK
    rows = [r0, r1, r2, r3]
    gs = [gs0, gs1, gs2, gs3]
    ss = [ss0, ss1, ss2, ss3]

    pltpu.sync_copy(trig_hbm, trig_v)
    pltpu.sync_copy(x_hbm.at[pl.ds(b0, BBLK)], idx_v)

    def issue_gather(j, b):
        pltpu.async_copy(table_hbm.at[idx_v.at[j]], rows[b], gs[b])

    def wait_gather(b):
        pltpu.make_async_copy(table_hbm.at[pl.ds(0, SEQ)], rows[b], gs[b]).wait()

    def wait_store(b):
        pltpu.make_async_copy(rows[b], out_hbm.at[pl.ds(0, SEQ)], ss[b]).wait()

    for b in range(NBUF):
        issue_gather(b, b)

    def outer(o, carry):
        for b in range(NBUF):
            j = o * NBUF + b
            wait_gather(b)

            def vstep(r, cy):
                for q in range(VECS_PER_ROW):
                    sl = pl.ds(q * LANES, LANES)
                    rows[b][r, sl] = rows[b][r, sl] * 8.0 + trig_v[r, sl]
                return cy

            lax.fori_loop(0, SEQ, vstep, 0, unroll=4)

            pltpu.async_copy(
                rows[b], out_hbm.at[pl.ds((b0 + j) * SEQ, SEQ)], ss[b])

            @pl.when(j + NBUF < BBLK)
            def _():
                wait_store(b)
                issue_gather(j + NBUF, b)
        return carry

    lax.fori_loop(0, BBLK // NBUF, outer, 0)
    for b in range(NBUF):
        wait_store(b)


@jax.jit
def kernel(x, table):
    trig8 = (_trig_table(EMB_DIM // 2, MAX_LEN)[:SEQ] * (EMB_DIM ** 0.5)
             ).astype(jnp.float32)
    xi = x.astype(jnp.int32)

    mesh = plsc.VectorSubcoreMesh(core_axis_name="c", subcore_axis_name="s")
    k = pl.kernel(
        _body,
        out_type=jax.ShapeDtypeStruct((BATCH * SEQ, EMB_DIM), jnp.float32),
        mesh=mesh,
        scratch_types=[
            pltpu.VMEM((BBLK, SEQ), jnp.int32),
            pltpu.VMEM((SEQ, EMB_DIM), jnp.float32),
            pltpu.VMEM((SEQ, EMB_DIM), jnp.float32),
            pltpu.VMEM((SEQ, EMB_DIM), jnp.float32),
            pltpu.VMEM((SEQ, EMB_DIM), jnp.float32),
            pltpu.VMEM((SEQ, EMB_DIM), jnp.float32),
            pltpu.SemaphoreType.DMA,
            pltpu.SemaphoreType.DMA,
            pltpu.SemaphoreType.DMA,
            pltpu.SemaphoreType.DMA,
            pltpu.SemaphoreType.DMA,
            pltpu.SemaphoreType.DMA,
            pltpu.SemaphoreType.DMA,
            pltpu.SemaphoreType.DMA,
        ],
        compiler_params=pltpu.CompilerParams(use_tc_tiling_on_sc=False),
    )
    out = k(xi, trig8, table)
    return out.reshape(BATCH, SEQ, EMB_DIM)


# speedup vs baseline: 1.3560x; 1.3560x over previous
import jax
import jax.numpy as jnp
from jax import lax
from jax.experimental import pallas as pl
from jax.experimental.pallas import tpu as pltpu
from jax.experimental.pallas import tpu_sc as plsc

VOCAB = 1000000
EMB_DIM = 64
MAX_LEN = 200
BATCH = 4096
SEQ = 200

NUM_CORES = 2
NUM_SUBCORES = 16
NUM_WORKERS = NUM_CORES * NUM_SUBCORES
BBLK = BATCH // NUM_WORKERS
LANES = 16
VECS_PER_ROW = EMB_DIM // LANES
NBUF = 4


def _trig_table(dim, max_len):
    len_size = jnp.tile(jnp.arange(max_len)[:, None], (1, dim)).astype(jnp.float32)
    dim_scale = jnp.power(10000.0, jnp.arange(dim).astype(jnp.float32) / dim)
    pos_s = jnp.sin(len_size / dim_scale)
    pos_c = jnp.cos(len_size / dim_scale)
    out = jnp.concatenate((pos_s, pos_c), axis=0)
    return out.reshape(max_len, -1)


def _body(xt_hbm, trig_hbm, table_hbm, out_hbm,
          idx_v, trig_v, r0, r1, r2, r3, gs0, gs1, gs2, gs3,
          ss0, ss1, ss2, ss3):
    c = lax.axis_index("c")
    s_ax = lax.axis_index("s")
    wid = s_ax * NUM_CORES + c
    b0 = wid * BBLK
    rows = [r0, r1, r2, r3]
    gs = [gs0, gs1, gs2, gs3]
    ss = [ss0, ss1, ss2, ss3]

    pltpu.sync_copy(trig_hbm, trig_v)
    pltpu.sync_copy(xt_hbm.at[:, pl.ds(b0, BBLK)], idx_v)

    def issue_gather(g, b):
        pltpu.async_copy(table_hbm.at[idx_v.at[g]], rows[b], gs[b])

    def wait_gather(b):
        pltpu.make_async_copy(table_hbm.at[pl.ds(0, BBLK)], rows[b], gs[b]).wait()

    def wait_store(b):
        pltpu.make_async_copy(rows[b], out_hbm.at[pl.ds(0, BBLK)], ss[b]).wait()

    for b in range(NBUF):
        issue_gather(b, b)

    def outer(o, carry):
        for b in range(NBUF):
            g = o * NBUF + b
            wait_gather(b)

            tvec = tuple(trig_v[g, pl.ds(q * LANES, LANES)]
                         for q in range(VECS_PER_ROW))

            def vstep(r, tv):
                for q in range(VECS_PER_ROW):
                    sl = pl.ds(q * LANES, LANES)
                    rows[b][r, sl] = rows[b][r, sl] * 8.0 + tv[q]
                return tv

            lax.fori_loop(0, BBLK, vstep, tvec, unroll=4)

            pltpu.async_copy(
                rows[b], out_hbm.at[pl.ds(g * BATCH + b0, BBLK)], ss[b])

            @pl.when(g + NBUF < SEQ)
            def _():
                wait_store(b)
                issue_gather(g + NBUF, b)
        return carry

    lax.fori_loop(0, SEQ // NBUF, outer, 0)
    for b in range(NBUF):
        wait_store(b)


@jax.jit
def kernel(x, table):
    trig8 = (_trig_table(EMB_DIM // 2, MAX_LEN)[:SEQ] * (EMB_DIM ** 0.5)
             ).astype(jnp.float32)
    xt = jnp.transpose(x).astype(jnp.int32)

    mesh = plsc.VectorSubcoreMesh(core_axis_name="c", subcore_axis_name="s")
    k = pl.kernel(
        _body,
        out_type=jax.ShapeDtypeStruct((SEQ * BATCH, EMB_DIM), jnp.float32),
        mesh=mesh,
        scratch_types=[
            pltpu.VMEM((SEQ, BBLK), jnp.int32),
            pltpu.VMEM((SEQ, EMB_DIM), jnp.float32),
            pltpu.VMEM((BBLK, EMB_DIM), jnp.float32),
            pltpu.VMEM((BBLK, EMB_DIM), jnp.float32),
            pltpu.VMEM((BBLK, EMB_DIM), jnp.float32),
            pltpu.VMEM((BBLK, EMB_DIM), jnp.float32),
            pltpu.SemaphoreType.DMA,
            pltpu.SemaphoreType.DMA,
            pltpu.SemaphoreType.DMA,
            pltpu.SemaphoreType.DMA,
            pltpu.SemaphoreType.DMA,
            pltpu.SemaphoreType.DMA,
            pltpu.SemaphoreType.DMA,
            pltpu.SemaphoreType.DMA,
        ],
        compiler_params=pltpu.CompilerParams(use_tc_tiling_on_sc=False),
    )
    out = k(xt, trig8, table)
    return out.reshape(SEQ, BATCH, EMB_DIM).transpose(1, 0, 2)


# speedup vs baseline: 1.6617x; 1.2255x over previous
import jax
import jax.numpy as jnp
from jax import lax
from jax.experimental import pallas as pl
from jax.experimental.pallas import tpu as pltpu
from jax.experimental.pallas import tpu_sc as plsc

VOCAB = 1000000
EMB_DIM = 64
MAX_LEN = 200
BATCH = 4096
SEQ = 200

NUM_CORES = 2
NUM_SUBCORES = 16
NUM_WORKERS = NUM_CORES * NUM_SUBCORES
BBLK = BATCH // NUM_WORKERS
LANES = 16
VECS_PER_ROW = EMB_DIM // LANES
NBUF = 4


def _trig_table(dim, max_len):
    len_size = jnp.tile(jnp.arange(max_len)[:, None], (1, dim)).astype(jnp.float32)
    dim_scale = jnp.power(10000.0, jnp.arange(dim).astype(jnp.float32) / dim)
    pos_s = jnp.sin(len_size / dim_scale)
    pos_c = jnp.cos(len_size / dim_scale)
    out = jnp.concatenate((pos_s, pos_c), axis=0)
    return out.reshape(max_len, -1)


def _body(xt_hbm, trig_hbm, table_hbm, out_hbm,
          idx_v, trig_v, r0, r1, r2, r3, gs0, gs1, gs2, gs3,
          ss0, ss1, ss2, ss3):
    c = lax.axis_index("c")
    s_ax = lax.axis_index("s")
    wid = s_ax * NUM_CORES + c
    b0 = wid * BBLK
    rows = [r0, r1, r2, r3]
    gs = [gs0, gs1, gs2, gs3]
    ss = [ss0, ss1, ss2, ss3]

    pltpu.sync_copy(trig_hbm, trig_v)
    pltpu.sync_copy(xt_hbm.at[:, pl.ds(b0, BBLK)], idx_v)

    def issue_gather(g, b):
        pltpu.async_copy(table_hbm.at[idx_v.at[g]], rows[b], gs[b])

    def wait_gather(b):
        pltpu.make_async_copy(table_hbm.at[pl.ds(0, BBLK)], rows[b], gs[b]).wait()

    def wait_store(b):
        pltpu.make_async_copy(rows[b], out_hbm.at[pl.ds(0, BBLK)], ss[b]).wait()

    for b in range(NBUF):
        issue_gather(b, b)

    def outer(o, carry):
        for b in range(NBUF):
            g = o * NBUF + b
            wait_gather(b)

            tvec = tuple(trig_v[g, pl.ds(q * LANES, LANES)]
                         for q in range(VECS_PER_ROW))

            def vstep(r, tv):
                for q in range(VECS_PER_ROW):
                    sl = pl.ds(q * LANES, LANES)
                    rows[b][r, sl] = rows[b][r, sl] * 8.0 + tv[q]
                return tv

            lax.fori_loop(0, BBLK, vstep, tvec, unroll=4)

            pltpu.async_copy(
                rows[b], out_hbm.at[pl.ds(g * BATCH + b0, BBLK)], ss[b])

            @pl.when(g + NBUF < SEQ)
            def _():
                wait_store(b)
                issue_gather(g + NBUF, b)
        return carry

    lax.fori_loop(0, SEQ // NBUF, outer, 0)
    for b in range(NBUF):
        wait_store(b)


@jax.jit
def kernel(x, table):
    trig8 = (_trig_table(EMB_DIM // 2, MAX_LEN)[:SEQ] * (EMB_DIM ** 0.5)
             ).astype(jnp.float32)
    xt = jnp.transpose(x).astype(jnp.int32)
    table2 = jnp.pad(table, ((0, 0), (0, EMB_DIM)))

    mesh = plsc.VectorSubcoreMesh(core_axis_name="c", subcore_axis_name="s")
    k = pl.kernel(
        _body,
        out_type=jax.ShapeDtypeStruct((SEQ * BATCH, 2 * EMB_DIM), jnp.float32),
        mesh=mesh,
        scratch_types=[
            pltpu.VMEM((SEQ, BBLK), jnp.int32),
            pltpu.VMEM((SEQ, EMB_DIM), jnp.float32),
            pltpu.VMEM((BBLK, 2 * EMB_DIM), jnp.float32),
            pltpu.VMEM((BBLK, 2 * EMB_DIM), jnp.float32),
            pltpu.VMEM((BBLK, 2 * EMB_DIM), jnp.float32),
            pltpu.VMEM((BBLK, 2 * EMB_DIM), jnp.float32),
            pltpu.SemaphoreType.DMA,
            pltpu.SemaphoreType.DMA,
            pltpu.SemaphoreType.DMA,
            pltpu.SemaphoreType.DMA,
            pltpu.SemaphoreType.DMA,
            pltpu.SemaphoreType.DMA,
            pltpu.SemaphoreType.DMA,
            pltpu.SemaphoreType.DMA,
        ],
        compiler_params=pltpu.CompilerParams(use_tc_tiling_on_sc=True),
    )
    out = k(xt, trig8, table2)
    return out[:, :EMB_DIM].reshape(SEQ, BATCH, EMB_DIM).transpose(1, 0, 2)
